# baseline (speedup 1.0000x reference)
import jax
import jax.numpy as jnp
from jax.experimental import pallas as pl
from jax.experimental.pallas import tpu as pltpu

_N_ANCHORS = 65536
_N_TARGETS = 256
_N_CLASSES = 21
_IOU_THRESH = 0.5
_BLOCK = 4096
_CHUNK = 512
_TSLAB = 8


def _loss_tile_kernel(tb_ref, tbt_ref, labels_ref, ant_ref, clst_ref, bpt_ref,
                      out_ref, iou_scr, oh_scr, row_scr):
    for j in range(_BLOCK // _CHUNK):
        sl = slice(j * _CHUNK, (j + 1) * _CHUNK)
        ax0 = jnp.broadcast_to(ant_ref[0:1, sl], (_TSLAB, _CHUNK))
        ay0 = jnp.broadcast_to(ant_ref[1:2, sl], (_TSLAB, _CHUNK))
        ax1 = jnp.broadcast_to(ant_ref[2:3, sl], (_TSLAB, _CHUNK))
        ay1 = jnp.broadcast_to(ant_ref[3:4, sl], (_TSLAB, _CHUNK))
        area_a = (ax1 - ax0) * (ay1 - ay0)

        mxs = []
        for i in range(_N_TARGETS // _TSLAB):
            r = slice(i * _TSLAB, (i + 1) * _TSLAB)
            tx0 = tb_ref[r, 0:1]
            ty0 = tb_ref[r, 1:2]
            tx1 = tb_ref[r, 2:3]
            ty1 = tb_ref[r, 3:4]
            area_te = (tx1 - tx0) * (ty1 - ty0) + 1e-8
            w = jnp.maximum(jnp.minimum(tx1, ax1) - jnp.maximum(tx0, ax0), 0.0)
            h = jnp.maximum(jnp.minimum(ty1, ay1) - jnp.maximum(ty0, ay0), 0.0)
            inter = w * h
            iou = inter / (area_te + area_a - inter)
            iou_scr[r, sl] = iou
            mxs.append(iou)
        while len(mxs) > 1:
            mxs = [jnp.maximum(mxs[k], mxs[k + 1]) if k + 1 < len(mxs)
                   else mxs[k] for k in range(0, len(mxs), 2)]
        row_scr[0:1, sl] = jnp.max(mxs[0], axis=0, keepdims=True)

    for j in range(_BLOCK // _CHUNK):
        sl = slice(j * _CHUNK, (j + 1) * _CHUNK)
        m = row_scr[0:1, sl]
        selv = jnp.where(m >= _IOU_THRESH, m, 2.0)
        sel_b = jnp.broadcast_to(selv, (_TSLAB, _CHUNK))
        for i in range(_N_TARGETS // _TSLAB):
            r = slice(i * _TSLAB, (i + 1) * _TSLAB)
            oh_scr[r, sl] = (iou_scr[r, sl] == sel_b).astype(jnp.float32)

    max_iou = row_scr[0:1, :]
    pos = (max_iou >= _IOU_THRESH).astype(jnp.float32)

    labels = labels_ref[...]
    cls_iota = jax.lax.broadcasted_iota(jnp.int32, (_N_CLASSES, _N_TARGETS), 0)
    labels_oh = (labels == cls_iota).astype(jnp.float32)
    payload = jnp.concatenate([tbt_ref[...], labels_oh], axis=0)
    gathered = jax.lax.dot_general(
        payload, oh_scr[...], (((1,), (0,)), ((), ())),
        preferred_element_type=jnp.float32)

    cls_t = gathered[4:, :]
    pred = clst_ref[...]
    ce = (jnp.maximum(pred, 0.0) - pred * cls_t
          + jnp.log1p(jnp.exp(-jnp.abs(pred))))
    p = jax.nn.sigmoid(pred)
    p_t = p * cls_t + (1.0 - p) * (1.0 - cls_t)
    a_t = 0.25 * cls_t + 0.75 * (1.0 - cls_t)
    om = 1.0 - p_t
    s_cls = jnp.sum(a_t * om * om * ce)

    ant = ant_ref[...]
    ax0 = ant[0:1, :]
    ay0 = ant[1:2, :]
    ax1 = ant[2:3, :]
    ay1 = ant[3:4, :]
    g = gathered[0:4, :]
    gx0 = g[0:1]
    gy0 = g[1:2]
    gx1 = g[2:3]
    gy1 = g[3:4]
    g_cx = (gx0 + gx1) * 0.5
    g_cy = (gy0 + gy1) * 0.5
    g_w = gx1 - gx0
    g_h = gy1 - gy0
    a_cx = (ax0 + ax1) * 0.5
    a_cy = (ay0 + ay1) * 0.5
    a_w = ax1 - ax0
    a_h = ay1 - ay0
    t_x = (g_cx - a_cx) / (a_w * 0.1)
    t_y = (g_cy - a_cy) / (a_h * 0.1)
    t_w = jnp.log(jnp.clip(g_w, 1e-6) / jnp.clip(a_w, 1e-6)) / 0.2
    t_h = jnp.log(jnp.clip(g_h, 1e-6) / jnp.clip(a_h, 1e-6)) / 0.2
    enc = jnp.concatenate([t_x, t_y, t_w, t_h], axis=0)
    bp = bpt_ref[...]
    d = jnp.abs(bp - enc)
    sl1 = jnp.where(d < 1.0, 0.5 * d * d, d - 0.5)
    s_reg = jnp.sum(sl1 * pos)
    s_pos = jnp.sum(pos)

    lane = jax.lax.broadcasted_iota(jnp.int32, (1, 1, 128), 2)
    out_ref[...] = (jnp.where(lane == 0, s_cls, 0.0)
                    + jnp.where(lane == 1, s_reg, 0.0)
                    + jnp.where(lane == 2, s_pos, 0.0))


def kernel(classification_preds, boxes_preds, anchors, target_boxes,
           target_labels):
    classification_preds = classification_preds.reshape(-1, _N_CLASSES)
    boxes_preds = boxes_preds.reshape(-1, 4)
    anchors = anchors.reshape(-1, 4)
    target_boxes = target_boxes.reshape(-1, 4)
    target_labels = target_labels.reshape(-1)

    labels = target_labels.astype(jnp.int32).reshape(1, _N_TARGETS)
    cls_t = classification_preds.T
    bp_t = boxes_preds.T
    an_t = anchors.T
    tbt = target_boxes.T

    n_blocks = _N_ANCHORS // _BLOCK
    out = pl.pallas_call(
        _loss_tile_kernel,
        grid=(n_blocks,),
        in_specs=[
            pl.BlockSpec((_N_TARGETS, 4), lambda i: (0, 0)),
            pl.BlockSpec((4, _N_TARGETS), lambda i: (0, 0)),
            pl.BlockSpec((1, _N_TARGETS), lambda i: (0, 0)),
            pl.BlockSpec((4, _BLOCK), lambda i: (0, i)),
            pl.BlockSpec((_N_CLASSES, _BLOCK), lambda i: (0, i)),
            pl.BlockSpec((4, _BLOCK), lambda i: (0, i)),
        ],
        out_specs=pl.BlockSpec((1, 1, 128), lambda i: (i, 0, 0)),
        out_shape=jax.ShapeDtypeStruct((n_blocks, 1, 128), jnp.float32),
        scratch_shapes=[
            pltpu.VMEM((_N_TARGETS, _BLOCK), jnp.float32),
            pltpu.VMEM((_N_TARGETS, _BLOCK), jnp.float32),
            pltpu.VMEM((1, _BLOCK), jnp.float32),
        ],
        compiler_params=pltpu.CompilerParams(
            dimension_semantics=("parallel",)),
    )(target_boxes, tbt, labels, an_t, cls_t, bp_t)

    sums = jnp.sum(out[:, 0, :], axis=0)
    s_cls = sums[0]
    s_reg = sums[1]
    pos_cnt = sums[2]
    cls_loss = s_cls / pos_cnt
    reg_loss = s_reg / (pos_cnt * 4.0)
    return (cls_loss + reg_loss, cls_loss, reg_loss)


# speedup vs baseline: 4.4344x; 4.4344x over previous
import jax
import jax.numpy as jnp
from jax.experimental import pallas as pl
from jax.experimental.pallas import tpu as pltpu

_N_ANCHORS = 65536
_N_TARGETS = 256
_N_CLASSES = 21
_IOU_THRESH = 0.5
_BLOCK = 16384


def _loss_tile_kernel(tb_ref, tbt_ref, labels_ref, ant_ref, clst_ref, bpt_ref,
                      out_ref):
    tb = tb_ref[...]
    ant = ant_ref[...]
    ax0 = ant[0:1, :]
    ay0 = ant[1:2, :]
    ax1 = ant[2:3, :]
    ay1 = ant[3:4, :]
    tx0 = tb[:, 0:1]
    ty0 = tb[:, 1:2]
    tx1 = tb[:, 2:3]
    ty1 = tb[:, 3:4]

    area_te = (tx1 - tx0) * (ty1 - ty0) + 1e-8
    area_a = (ax1 - ax0) * (ay1 - ay0)
    w = jnp.clip(jnp.minimum(tx1, ax1) - jnp.maximum(tx0, ax0), 0.0)
    h = jnp.clip(jnp.minimum(ty1, ay1) - jnp.maximum(ty0, ay0), 0.0)
    inter = w * h
    iou = inter / (area_te + area_a - inter)

    max_iou = jnp.max(iou, axis=0, keepdims=True)
    pos_b = max_iou >= _IOU_THRESH
    pos = pos_b.astype(jnp.float32)
    sel_max = jnp.where(pos_b, max_iou, 2.0)
    onehot = (iou == sel_max).astype(jnp.float32)

    labels = labels_ref[...]
    cls_iota = jax.lax.broadcasted_iota(jnp.int32, (_N_CLASSES, _N_TARGETS), 0)
    labels_oh = (labels == cls_iota).astype(jnp.float32)
    payload = jnp.concatenate([tbt_ref[...], labels_oh], axis=0)
    gathered = jax.lax.dot_general(
        payload, onehot, (((1,), (0,)), ((), ())),
        preferred_element_type=jnp.float32)

    cls_t = gathered[4:, :]
    pred = clst_ref[...]
    ce = (jnp.maximum(pred, 0.0) - pred * cls_t
          + jnp.log1p(jnp.exp(-jnp.abs(pred))))
    p = jax.nn.sigmoid(pred)
    p_t = p * cls_t + (1.0 - p) * (1.0 - cls_t)
    a_t = 0.25 * cls_t + 0.75 * (1.0 - cls_t)
    om = 1.0 - p_t
    s_cls = jnp.sum(a_t * om * om * ce)

    g = gathered[0:4, :]
    gx0 = g[0:1]
    gy0 = g[1:2]
    gx1 = g[2:3]
    gy1 = g[3:4]
    g_cx = (gx0 + gx1) * 0.5
    g_cy = (gy0 + gy1) * 0.5
    g_w = gx1 - gx0
    g_h = gy1 - gy0
    a_cx = (ax0 + ax1) * 0.5
    a_cy = (ay0 + ay1) * 0.5
    a_w = ax1 - ax0
    a_h = ay1 - ay0
    t_x = (g_cx - a_cx) / (a_w * 0.1)
    t_y = (g_cy - a_cy) / (a_h * 0.1)
    t_w = jnp.log(jnp.clip(g_w, 1e-6) / jnp.clip(a_w, 1e-6)) / 0.2
    t_h = jnp.log(jnp.clip(g_h, 1e-6) / jnp.clip(a_h, 1e-6)) / 0.2
    enc = jnp.concatenate([t_x, t_y, t_w, t_h], axis=0)
    bp = bpt_ref[...]
    d = jnp.abs(bp - enc)
    sl1 = jnp.where(d < 1.0, 0.5 * d * d, d - 0.5)
    s_reg = jnp.sum(sl1 * pos)
    s_pos = jnp.sum(pos)

    lane = jax.lax.broadcasted_iota(jnp.int32, (1, 1, 128), 2)
    out_ref[...] = (jnp.where(lane == 0, s_cls, 0.0)
                    + jnp.where(lane == 1, s_reg, 0.0)
                    + jnp.where(lane == 2, s_pos, 0.0))


def kernel(classification_preds, boxes_preds, anchors, target_boxes,
           target_labels):
    classification_preds = classification_preds.reshape(-1, _N_CLASSES)
    boxes_preds = boxes_preds.reshape(-1, 4)
    anchors = anchors.reshape(-1, 4)
    target_boxes = target_boxes.reshape(-1, 4)
    target_labels = target_labels.reshape(-1)

    labels = target_labels.astype(jnp.int32).reshape(1, _N_TARGETS)
    cls_t = classification_preds.T
    bp_t = boxes_preds.T
    an_t = anchors.T
    tbt = target_boxes.T

    n_blocks = _N_ANCHORS // _BLOCK
    out = pl.pallas_call(
        _loss_tile_kernel,
        grid=(n_blocks,),
        in_specs=[
            pl.BlockSpec((_N_TARGETS, 4), lambda i: (0, 0)),
            pl.BlockSpec((4, _N_TARGETS), lambda i: (0, 0)),
            pl.BlockSpec((1, _N_TARGETS), lambda i: (0, 0)),
            pl.BlockSpec((4, _BLOCK), lambda i: (0, i)),
            pl.BlockSpec((_N_CLASSES, _BLOCK), lambda i: (0, i)),
            pl.BlockSpec((4, _BLOCK), lambda i: (0, i)),
        ],
        out_specs=pl.BlockSpec((1, 1, 128), lambda i: (i, 0, 0)),
        out_shape=jax.ShapeDtypeStruct((n_blocks, 1, 128), jnp.float32),
        compiler_params=pltpu.CompilerParams(
            dimension_semantics=("parallel",)),
    )(target_boxes, tbt, labels, an_t, cls_t, bp_t)

    sums = jnp.sum(out[:, 0, :], axis=0)
    s_cls = sums[0]
    s_reg = sums[1]
    pos_cnt = sums[2]
    cls_loss = s_cls / pos_cnt
    reg_loss = s_reg / (pos_cnt * 4.0)
    return (cls_loss + reg_loss, cls_loss, reg_loss)
